# Initial kernel scaffold; baseline (speedup 1.0000x reference)
import functools

import jax
import jax.numpy as jnp
from jax import lax
from jax.experimental import pallas as pl
from jax.experimental.pallas import tpu as pltpu
from jax.experimental.pallas import tpu_sc as plsc

NC, NS, LANES = 2, 16, 16
NW = NC * NS


def _sc_gather_fm(feat_flat, emb, bias_flat, B, F, D):
    EPW = B // NW
    CH = 128
    NCH = EPW // CH
    RPC = CH * F
    IPW = EPW * F
    G = 128
    NG = RPC // G

    mesh = plsc.VectorSubcoreMesh(core_axis_name="c", subcore_axis_name="s")

    @functools.partial(
        pl.kernel,
        out_type=(
            jax.ShapeDtypeStruct((B, D), jnp.float32),
            jax.ShapeDtypeStruct((B,), jnp.float32),
        ),
        mesh=mesh,
        scratch_types=[
            pltpu.VMEM((IPW,), jnp.int32),
            pltpu.VMEM((RPC, D), jnp.float32),
            pltpu.VMEM((RPC,), jnp.float32),
            pltpu.VMEM((CH, D), jnp.float32),
            pltpu.VMEM((CH,), jnp.float32),
            pltpu.SemaphoreType.DMA,
            pltpu.SemaphoreType.DMA,
        ],
    )
    def k(feat_hbm, emb_hbm, bias_hbm, fm_hbm, fb_hbm,
          idx_v, rows_v, bias_v, fm_v, fb_v, sem_r, sem_b):
        wid = lax.axis_index("s") * NC + lax.axis_index("c")
        ebase = wid * EPW
        pltpu.sync_copy(feat_hbm.at[pl.ds(ebase * F, IPW)], idx_v)
        lanes = lax.iota(jnp.int32, LANES) * F
        for c in range(NCH):
            ioff = c * RPC
            copies = []
            for g in range(NG):
                sl = pl.ds(ioff + g * G, G)
                copies.append(pltpu.async_copy(
                    emb_hbm.at[idx_v.at[sl]], rows_v.at[pl.ds(g * G, G), :], sem_r))
                copies.append(pltpu.async_copy(
                    bias_hbm.at[idx_v.at[sl]], bias_v.at[pl.ds(g * G, G)], sem_b))
            for cp in copies:
                cp.wait()

            def elem(i, _):
                r0 = i * F
                v = rows_v[r0]
                acc = v
                accsq = v * v
                for f in range(1, F):
                    v = rows_v[r0 + f]
                    acc = acc + v
                    accsq = accsq + v * v
                fm_v[i] = 0.5 * (acc * acc - accsq)
                return 0

            lax.fori_loop(0, CH, elem, 0)

            for grp in range(CH // LANES):
                base = grp * LANES * F
                bacc = plsc.load_gather(bias_v, [lanes + base])
                for f in range(1, F):
                    bacc = bacc + plsc.load_gather(bias_v, [lanes + (base + f)])
                fb_v[pl.ds(grp * LANES, LANES)] = bacc

            pltpu.sync_copy(fm_v, fm_hbm.at[pl.ds(ebase + c * CH, CH), :])
            pltpu.sync_copy(fb_v, fb_hbm.at[pl.ds(ebase + c * CH, CH)])

    return k(feat_flat, emb, bias_flat)


def _tc_mlp(fm, fb, W1, b1, W2, b2, Wp, bp, Wb):
    B, D = fm.shape
    BLK = 2048

    def body(x_ref, fb_ref, W1_ref, b1_ref, W2_ref, b2_ref, Wp_ref, bp_ref,
             Wb_ref, o_ref):
        x = x_ref[...]
        h = jnp.maximum(
            jnp.dot(x, W1_ref[...], preferred_element_type=jnp.float32)
            + b1_ref[...], 0.0)
        h = jnp.maximum(
            jnp.dot(h, W2_ref[...], preferred_element_type=jnp.float32)
            + b2_ref[...], 0.0)
        o = (jnp.dot(h, Wp_ref[...], preferred_element_type=jnp.float32)
             + bp_ref[...] + fb_ref[...] + Wb_ref[...])
        o_ref[...] = o

    full = lambda a: pl.BlockSpec(a.shape, lambda i: (0, 0))
    return pl.pallas_call(
        body,
        grid=(B // BLK,),
        in_specs=[
            pl.BlockSpec((BLK, D), lambda i: (i, 0)),
            pl.BlockSpec((BLK, 1), lambda i: (i, 0)),
            full(W1), full(b1), full(W2), full(b2), full(Wp), full(bp), full(Wb),
        ],
        out_specs=pl.BlockSpec((BLK, 1), lambda i: (i, 0)),
        out_shape=jax.ShapeDtypeStruct((B, 1), jnp.float32),
    )(fm, fb, W1, b1, W2, b2, Wp, bp, Wb)


def kernel(features, labels, emb, bias_table, W_bias, W1, b1, W2, b2, Wp, bp):
    B, F = features.shape
    M, D = emb.shape
    feat_flat = features.reshape(B * F)
    bias_flat = bias_table.reshape(M)
    fm, fbias = _sc_gather_fm(feat_flat, emb, bias_flat, B, F, D)
    return _tc_mlp(fm, fbias.reshape(B, 1), W1, b1.reshape(1, -1), W2,
                   b2.reshape(1, -1), Wp, bp.reshape(1, 1), W_bias)


# speedup vs baseline: 1.2467x; 1.2467x over previous
import functools

import jax
import jax.numpy as jnp
from jax import lax
from jax.experimental import pallas as pl
from jax.experimental.pallas import tpu as pltpu
from jax.experimental.pallas import tpu_sc as plsc

NC, NS, LANES = 2, 16, 16
NW = NC * NS


def _sc_gather_fm(feat_flat, emb, bias_flat, B, F, D):
    EPW = B // NW
    CH = 128
    NCH = EPW // CH
    RPC = CH * F
    IPW = EPW * F
    G = 128
    NG = RPC // G

    mesh = plsc.VectorSubcoreMesh(core_axis_name="c", subcore_axis_name="s")

    @functools.partial(
        pl.kernel,
        out_type=(
            jax.ShapeDtypeStruct((B, D), jnp.float32),
            jax.ShapeDtypeStruct((B * F,), jnp.float32),
        ),
        mesh=mesh,
        scratch_types=[
            pltpu.VMEM((IPW,), jnp.int32),
            pltpu.VMEM((RPC, D), jnp.float32),
            pltpu.VMEM((RPC,), jnp.float32),
            pltpu.VMEM((CH, D), jnp.float32),
            pltpu.SemaphoreType.DMA,
            pltpu.SemaphoreType.DMA,
        ],
        compiler_params=pltpu.CompilerParams(use_tc_tiling_on_sc=False),
    )
    def k(feat_hbm, emb_hbm, bias_hbm, fm_hbm, bv_hbm,
          idx_v, rows_v, bias_v, fm_v, sem_r, sem_b):
        wid = lax.axis_index("s") * NC + lax.axis_index("c")
        ebase = wid * EPW
        pltpu.sync_copy(feat_hbm.at[pl.ds(ebase * F, IPW)], idx_v)
        for c in range(NCH):
            ioff = c * RPC
            copies = []
            for g in range(NG):
                sl = pl.ds(ioff + g * G, G)
                copies.append(pltpu.async_copy(
                    emb_hbm.at[idx_v.at[sl]], rows_v.at[pl.ds(g * G, G), :], sem_r))
                copies.append(pltpu.async_copy(
                    bias_hbm.at[idx_v.at[sl]], bias_v.at[pl.ds(g * G, G)], sem_b))
            for cp in copies:
                cp.wait()

            def elem(i, _):
                r0 = i * F
                v = rows_v[r0]
                acc = v
                accsq = v * v
                for f in range(1, F):
                    v = rows_v[r0 + f]
                    acc = acc + v
                    accsq = accsq + v * v
                fm_v[i] = 0.5 * (acc * acc - accsq)
                return 0

            lax.fori_loop(0, CH, elem, 0)

            pltpu.sync_copy(fm_v, fm_hbm.at[pl.ds(ebase + c * CH, CH), :])
            pltpu.sync_copy(bias_v, bv_hbm.at[pl.ds((ebase + c * CH) * F, RPC)])

    return k(feat_flat, emb, bias_flat)


def _tc_mlp(fm, bv, W1, b1, W2, b2, Wp, bp, Wb):
    B, D = fm.shape
    F = bv.shape[1]
    BLK = 2048

    def body(x_ref, bv_ref, W1_ref, b1_ref, W2_ref, b2_ref, Wp_ref, bp_ref,
             Wb_ref, o_ref):
        x = x_ref[...]
        h = jnp.maximum(
            jnp.dot(x, W1_ref[...], preferred_element_type=jnp.float32)
            + b1_ref[...], 0.0)
        h = jnp.maximum(
            jnp.dot(h, W2_ref[...], preferred_element_type=jnp.float32)
            + b2_ref[...], 0.0)
        fb = jnp.sum(bv_ref[...], axis=1, keepdims=True)
        o = (jnp.dot(h, Wp_ref[...], preferred_element_type=jnp.float32)
             + bp_ref[...] + fb + Wb_ref[...])
        o_ref[...] = o

    full = lambda a: pl.BlockSpec(a.shape, lambda i: (0, 0))
    return pl.pallas_call(
        body,
        grid=(B // BLK,),
        in_specs=[
            pl.BlockSpec((BLK, D), lambda i: (i, 0)),
            pl.BlockSpec((BLK, F), lambda i: (i, 0)),
            full(W1), full(b1), full(W2), full(b2), full(Wp), full(bp), full(Wb),
        ],
        out_specs=pl.BlockSpec((BLK, 1), lambda i: (i, 0)),
        out_shape=jax.ShapeDtypeStruct((B, 1), jnp.float32),
    )(fm, bv, W1, b1, W2, b2, Wp, bp, Wb)


def kernel(features, labels, emb, bias_table, W_bias, W1, b1, W2, b2, Wp, bp):
    B, F = features.shape
    M, D = emb.shape
    feat_flat = features.reshape(B * F)
    bias_flat = bias_table.reshape(M)
    fm, bvals = _sc_gather_fm(feat_flat, emb, bias_flat, B, F, D)
    return _tc_mlp(fm, bvals.reshape(B, F), W1, b1.reshape(1, -1), W2,
                   b2.reshape(1, -1), Wp, bp.reshape(1, 1), W_bias)
